# baseline (speedup 1.0000x reference)
import jax
import jax.numpy as jnp
from jax import lax
from jax.experimental import pallas as pl
from jax.experimental.pallas import tpu as pltpu
from jax.experimental.pallas import tpu_sc as plsc

_N_NODES = 10000
_N_EDGES = 320000
_D_FEAT = 128
_D_EDGE = 16
_D_OUT = 16

_NC, _NS = 2, 16
_NW = _NC * _NS

_CT = _N_EDGES // 128
_CT_PAD = 2560
_N_PAD = _CT_PAD * 128
_EPW = _N_PAD // _NW
_CHUNK = 1024
_NCHUNK = _EPW // _CHUNK
_TPC = _CHUNK // 128

_CTB = 128
_GRID_B = 20


def _nodeproj_body(nf_ref, ws_ref, wd_ref, ps_ref, pd_ref):
    nf = nf_ref[...]
    ps_ref[...] = jnp.dot(nf, ws_ref[...], preferred_element_type=jnp.float32)
    pd_ref[...] = jnp.dot(nf, wd_ref[...], preferred_element_type=jnp.float32)


def _sc_gather_body(ps_hbm, pd_hbm, si_hbm, di_hbm, gt_hbm,
                    idx_s0, idx_s1, idx_d0, idx_d1,
                    buf_a0, buf_a1, buf_b0, buf_b1, buf_t,
                    sem_i0, sem_i1, sem_a0, sem_a1, sem_b0, sem_b1):
    wid = lax.axis_index("s") * _NC + lax.axis_index("c")
    base = wid * _EPW
    lanes = jnp.arange(16, dtype=jnp.int32)
    idx_s, idx_d = [idx_s0, idx_s1], [idx_d0, idx_d1]
    buf_a, buf_b = [buf_a0, buf_a1], [buf_b0, buf_b1]
    sem_i, sem_a, sem_b = [sem_i0, sem_i1], [sem_a0, sem_a1], [sem_b0, sem_b1]

    def start_idx(c, p):
        off = base + c * _CHUNK
        ci = pltpu.async_copy(si_hbm.at[pl.ds(off, _CHUNK)], idx_s[p], sem_i[p])
        cd = pltpu.async_copy(di_hbm.at[pl.ds(off, _CHUNK)], idx_d[p], sem_i[p])
        return ci, cd

    def start_gathers(p):
        ga = pltpu.async_copy(ps_hbm.at[idx_s[p]], buf_a[p], sem_a[p])
        gb = pltpu.async_copy(pd_hbm.at[idx_d[p]], buf_b[p], sem_b[p])
        return ga, gb

    i_cur = start_idx(0, 0)
    i_cur[0].wait()
    i_cur[1].wait()
    g = [None, None]
    g[0] = start_gathers(0)
    for c in range(_NCHUNK):
        p = c & 1
        if c + 1 < _NCHUNK:
            i_nxt = start_idx(c + 1, p ^ 1)
        g[p][0].wait()
        g[p][1].wait()
        if c + 1 < _NCHUNK:
            i_nxt[0].wait()
            i_nxt[1].wait()
            g[p ^ 1] = start_gathers(p ^ 1)
        ba, bb = buf_a[p], buf_b[p]
        zeros16 = jnp.full((16,), 0, jnp.int32)
        for tt in range(_TPC):
            rows_t = lanes + tt * 16

            def col_body(cc, carry2, ba=ba, bb=bb, rows_t=rows_t, tt=tt):
                c4 = cc * 4
                for k in range(4):
                    e = tt * 128 + c4 + k
                    v = ba[e, :] + bb[e, :]
                    plsc.store_scatter(buf_t, [rows_t, zeros16 + (c4 + k)], v)
                return carry2

            lax.fori_loop(0, 128 // 4, col_body, 0)
        pltpu.sync_copy(buf_t,
                        gt_hbm.at[pl.ds((base + c * _CHUNK) // 8, _TPC * 16)])


def _edge_body(wet_ref, ef_ref, g_ref, b_ref, o_ref):
    wet = wet_ref[...]
    bb = b_ref[...]
    mm = jnp.dot(wet, ef_ref[...], preferred_element_type=jnp.float32) + bb
    for t in range(_CTB):
        o_ref[:, pl.ds(t * 128, 128)] = (
            mm[:, t * 128:(t + 1) * 128] + g_ref[t])


def kernel(node_feats, edge_feats, edge_indices, W, b):
    Wt = W.T
    wet = Wt[:, :_D_EDGE]
    Ws = W[_D_EDGE:_D_EDGE + _D_FEAT]
    Wd = W[_D_EDGE + _D_FEAT:]

    ps, pd = pl.pallas_call(
        _nodeproj_body,
        out_shape=[jax.ShapeDtypeStruct((_N_NODES, _D_OUT), jnp.float32)] * 2,
    )(node_feats, Ws, Wd)

    sc_gather = pl.kernel(
        _sc_gather_body,
        out_type=jax.ShapeDtypeStruct((_CT_PAD * _D_OUT, 128), jnp.float32),
        mesh=plsc.VectorSubcoreMesh(core_axis_name="c", subcore_axis_name="s"),
        scratch_types=(
            [pltpu.VMEM((_CHUNK,), jnp.int32)] * 4
            + [pltpu.VMEM((_CHUNK, _D_OUT), jnp.float32)] * 4
            + [pltpu.VMEM((_TPC * _D_OUT, 128), jnp.float32)]
            + [pltpu.SemaphoreType.DMA] * 6
        ),
        compiler_params=pltpu.CompilerParams(use_tc_tiling_on_sc=False,
                                             needs_layout_passes=False),
    )
    si = jnp.pad(edge_indices[0], (0, _N_PAD - _N_EDGES))
    di = jnp.pad(edge_indices[1], (0, _N_PAD - _N_EDGES))
    gt = sc_gather(ps, pd, si, di)

    g3 = gt.reshape(_CT_PAD, _D_OUT, 128)
    ef_t = edge_feats.T
    b_col = b.reshape(_D_OUT, 1)
    out_t = pl.pallas_call(
        _edge_body,
        grid=(_GRID_B,),
        in_specs=[
            pl.BlockSpec((_D_OUT, _D_EDGE), lambda i: (0, 0)),
            pl.BlockSpec((_D_OUT, 128 * _CTB), lambda i: (0, i)),
            pl.BlockSpec((_CTB, _D_OUT, 128), lambda i: (i, 0, 0)),
            pl.BlockSpec((_D_OUT, 1), lambda i: (0, 0)),
        ],
        out_specs=pl.BlockSpec((_D_OUT, 128 * _CTB), lambda i: (0, i)),
        out_shape=jax.ShapeDtypeStruct((_D_OUT, _N_EDGES), jnp.float32),
    )(wet, ef_t, g3, b_col)
    return out_t.T


# speedup vs baseline: 1.2307x; 1.2307x over previous
import jax
import jax.numpy as jnp
from jax import lax
from jax.experimental import pallas as pl
from jax.experimental.pallas import tpu as pltpu
from jax.experimental.pallas import tpu_sc as plsc

_N_NODES = 10000
_N_EDGES = 320000
_D_FEAT = 128
_D_EDGE = 16
_D_OUT = 16

_NC, _NS = 2, 16
_NW = _NC * _NS

_CT = _N_EDGES // 128
_CT_PAD = 2560
_N_PAD = _CT_PAD * 128
_EPW = _N_PAD // _NW
_CHUNK = 1024
_NCHUNK = _EPW // _CHUNK
_TPC = _CHUNK // 128

_CTB = 128
_GRID_B = 20


def _nodeproj_body(nf_ref, ws_ref, wd_ref, ps_ref, pd_ref):
    nf = nf_ref[...]
    ps_ref[...] = jnp.dot(nf, ws_ref[...], preferred_element_type=jnp.float32)
    pd_ref[...] = jnp.dot(nf, wd_ref[...], preferred_element_type=jnp.float32)


def _sc_gather_body(ps_hbm, pd_hbm, si_hbm, di_hbm, gt_hbm,
                    idx_s0, idx_s1, idx_d0, idx_d1,
                    buf_a0, buf_a1, buf_b0, buf_b1, buf_t,
                    sem_i0, sem_i1, sem_a0, sem_a1, sem_b0, sem_b1):
    wid = lax.axis_index("s") * _NC + lax.axis_index("c")
    base = wid * _EPW
    lanes = jnp.arange(16, dtype=jnp.int32)
    idx_s, idx_d = [idx_s0, idx_s1], [idx_d0, idx_d1]
    buf_a, buf_b = [buf_a0, buf_a1], [buf_b0, buf_b1]
    sem_i, sem_a, sem_b = [sem_i0, sem_i1], [sem_a0, sem_a1], [sem_b0, sem_b1]

    def start_idx(c, p):
        off = base + c * _CHUNK
        ci = pltpu.async_copy(si_hbm.at[pl.ds(off, _CHUNK)], idx_s[p], sem_i[p])
        cd = pltpu.async_copy(di_hbm.at[pl.ds(off, _CHUNK)], idx_d[p], sem_i[p])
        return ci, cd

    def start_gathers(p):
        ga = pltpu.async_copy(ps_hbm.at[idx_s[p]], buf_a[p], sem_a[p])
        gb = pltpu.async_copy(pd_hbm.at[idx_d[p]], buf_b[p], sem_b[p])
        return ga, gb

    i_cur = start_idx(0, 0)
    i_cur[0].wait()
    i_cur[1].wait()
    g = [None, None]
    g[0] = start_gathers(0)
    for c in range(_NCHUNK):
        p = c & 1
        if c + 1 < _NCHUNK:
            i_nxt = start_idx(c + 1, p ^ 1)
        g[p][0].wait()
        g[p][1].wait()
        if c + 1 < _NCHUNK:
            i_nxt[0].wait()
            i_nxt[1].wait()
            g[p ^ 1] = start_gathers(p ^ 1)
        ba, bb = buf_a[p], buf_b[p]
        zeros16 = jnp.full((16,), 0, jnp.int32)
        for tt in range(_TPC):
            rows_t = lanes + tt * 16

            def col_body(cc, carry2, ba=ba, bb=bb, rows_t=rows_t, tt=tt):
                c4 = cc * 4
                for k in range(4):
                    e = tt * 128 + c4 + k
                    v = ba[e, :] + bb[e, :]
                    plsc.store_scatter(buf_t, [rows_t, zeros16 + (c4 + k)], v)
                return carry2

            lax.fori_loop(0, 128 // 4, col_body, 0)
        pltpu.sync_copy(buf_t.at[:, pl.ds(0, 128)],
                        gt_hbm.at[pl.ds((base + c * _CHUNK) // 8, _TPC * 16)])


def _edge_body(wet_ref, ef_ref, g_ref, b_ref, o_ref):
    wet = wet_ref[...]
    bb = b_ref[...]
    mm = jnp.dot(wet, ef_ref[...], preferred_element_type=jnp.float32) + bb
    for t in range(_CTB):
        o_ref[:, pl.ds(t * 128, 128)] = (
            mm[:, t * 128:(t + 1) * 128] + g_ref[t])


def kernel(node_feats, edge_feats, edge_indices, W, b):
    Wt = W.T
    wet = Wt[:, :_D_EDGE]
    Ws = W[_D_EDGE:_D_EDGE + _D_FEAT]
    Wd = W[_D_EDGE + _D_FEAT:]

    ps, pd = pl.pallas_call(
        _nodeproj_body,
        out_shape=[jax.ShapeDtypeStruct((_N_NODES, _D_OUT), jnp.float32)] * 2,
    )(node_feats, Ws, Wd)

    sc_gather = pl.kernel(
        _sc_gather_body,
        out_type=jax.ShapeDtypeStruct((_CT_PAD * _D_OUT, 128), jnp.float32),
        mesh=plsc.VectorSubcoreMesh(core_axis_name="c", subcore_axis_name="s"),
        scratch_types=(
            [pltpu.VMEM((_CHUNK,), jnp.int32)] * 4
            + [pltpu.VMEM((_CHUNK, _D_OUT), jnp.float32)] * 4
            + [pltpu.VMEM((_TPC * _D_OUT, 129), jnp.float32)]
            + [pltpu.SemaphoreType.DMA] * 6
        ),
        compiler_params=pltpu.CompilerParams(use_tc_tiling_on_sc=False,
                                             needs_layout_passes=False),
    )
    si = jnp.pad(edge_indices[0], (0, _N_PAD - _N_EDGES))
    di = jnp.pad(edge_indices[1], (0, _N_PAD - _N_EDGES))
    gt = sc_gather(ps, pd, si, di)

    g3 = gt.reshape(_CT_PAD, _D_OUT, 128)
    ef_t = edge_feats.T
    b_col = b.reshape(_D_OUT, 1)
    out_t = pl.pallas_call(
        _edge_body,
        grid=(_GRID_B,),
        in_specs=[
            pl.BlockSpec((_D_OUT, _D_EDGE), lambda i: (0, 0)),
            pl.BlockSpec((_D_OUT, 128 * _CTB), lambda i: (0, i)),
            pl.BlockSpec((_CTB, _D_OUT, 128), lambda i: (i, 0, 0)),
            pl.BlockSpec((_D_OUT, 1), lambda i: (0, 0)),
        ],
        out_specs=pl.BlockSpec((_D_OUT, 128 * _CTB), lambda i: (0, i)),
        out_shape=jax.ShapeDtypeStruct((_D_OUT, _N_EDGES), jnp.float32),
    )(wet, ef_t, g3, b_col)
    return out_t.T


# speedup vs baseline: 1.4546x; 1.1819x over previous
import jax
import jax.numpy as jnp
from jax import lax
from jax.experimental import pallas as pl
from jax.experimental.pallas import tpu as pltpu
from jax.experimental.pallas import tpu_sc as plsc

_N_NODES = 10000
_N_EDGES = 320000
_D_FEAT = 128
_D_EDGE = 16
_D_OUT = 16

_NC, _NS = 2, 16
_NW = _NC * _NS
_EPW = _N_EDGES // _NW
_CHUNK = 1000
_NCHUNK = _EPW // _CHUNK

_CT = _N_EDGES // 128
_CT_PAD = 2560
_N_PAD = _CT_PAD * 128
_CTB = 128
_GRID_B = 20


def _nodeproj_body(nf_ref, ws_ref, wd_ref, ps_ref, pd_ref):
    nf = nf_ref[...]
    ps_ref[...] = jnp.dot(nf, ws_ref[...], preferred_element_type=jnp.float32)
    pd_ref[...] = jnp.dot(nf, wd_ref[...], preferred_element_type=jnp.float32)


def _sc_gather_body(ps_hbm, pd_hbm, si_hbm, di_hbm, gt_hbm,
                    idx_s0, idx_s1, idx_d0, idx_d1,
                    buf_a0, buf_a1, buf_b0, buf_b1, buf_t,
                    sem_i0, sem_i1, sem_a0, sem_a1, sem_b0, sem_b1):
    wid = lax.axis_index("s") * _NC + lax.axis_index("c")
    base = wid * _EPW
    lanes = jnp.arange(16, dtype=jnp.int32)
    idx_s, idx_d = [idx_s0, idx_s1], [idx_d0, idx_d1]
    buf_a, buf_b = [buf_a0, buf_a1], [buf_b0, buf_b1]
    sem_i, sem_a, sem_b = [sem_i0, sem_i1], [sem_a0, sem_a1], [sem_b0, sem_b1]

    def start_idx(c, p):
        off = base + c * _CHUNK
        ci = pltpu.async_copy(si_hbm.at[pl.ds(off, _CHUNK)], idx_s[p], sem_i[p])
        cd = pltpu.async_copy(di_hbm.at[pl.ds(off, _CHUNK)], idx_d[p], sem_i[p])
        return ci, cd

    def start_gathers(p):
        ga = pltpu.async_copy(ps_hbm.at[idx_s[p]], buf_a[p], sem_a[p])
        gb = pltpu.async_copy(pd_hbm.at[idx_d[p]], buf_b[p], sem_b[p])
        return ga, gb

    i_cur = start_idx(0, 0)
    i_cur[0].wait()
    i_cur[1].wait()
    g = [None, None]
    g[0] = start_gathers(0)
    for c in range(_NCHUNK):
        p = c & 1
        if c + 1 < _NCHUNK:
            i_nxt = start_idx(c + 1, p ^ 1)
        g[p][0].wait()
        g[p][1].wait()
        if c + 1 < _NCHUNK:
            i_nxt[0].wait()
            i_nxt[1].wait()
            g[p ^ 1] = start_gathers(p ^ 1)
        ba, bb = buf_a[p], buf_b[p]

        def row_body(r, carry2, ba=ba, bb=bb):
            r4 = r * 4
            for k in range(4):
                v = ba[r4 + k, :] + bb[r4 + k, :]
                plsc.store_scatter(
                    buf_t, [lanes, jnp.full((16,), 0, jnp.int32) + (r4 + k)], v)
            return carry2

        lax.fori_loop(0, _CHUNK // 4, row_body, 0)
        off = base + c * _CHUNK
        pltpu.sync_copy(buf_t, gt_hbm.at[:, pl.ds(off, _CHUNK)])


def _edge_body(wet_ref, ef_ref, g_ref, b_ref, o_ref):
    wet = wet_ref[...]
    bb = b_ref[...]
    mm = jnp.dot(wet, ef_ref[...], preferred_element_type=jnp.float32) + bb
    for t in range(_CTB):
        o_ref[:, pl.ds(t * 128, 128)] = (
            mm[:, t * 128:(t + 1) * 128] + g_ref[:, t, :])


def kernel(node_feats, edge_feats, edge_indices, W, b):
    Wt = W.T
    wet = Wt[:, :_D_EDGE]
    Ws = W[_D_EDGE:_D_EDGE + _D_FEAT]
    Wd = W[_D_EDGE + _D_FEAT:]

    ps, pd = pl.pallas_call(
        _nodeproj_body,
        out_shape=[jax.ShapeDtypeStruct((_N_NODES, _D_OUT), jnp.float32)] * 2,
    )(node_feats, Ws, Wd)

    sc_gather = pl.kernel(
        _sc_gather_body,
        out_type=jax.ShapeDtypeStruct((_D_OUT, _N_PAD), jnp.float32),
        mesh=plsc.VectorSubcoreMesh(core_axis_name="c", subcore_axis_name="s"),
        scratch_types=(
            [pltpu.VMEM((_CHUNK,), jnp.int32)] * 4
            + [pltpu.VMEM((_CHUNK, _D_OUT), jnp.float32)] * 4
            + [pltpu.VMEM((_D_OUT, _CHUNK), jnp.float32)]
            + [pltpu.SemaphoreType.DMA] * 6
        ),
        compiler_params=pltpu.CompilerParams(use_tc_tiling_on_sc=False,
                                             needs_layout_passes=False),
    )
    gt = sc_gather(ps, pd, edge_indices[0], edge_indices[1])

    g3 = gt.reshape(_D_OUT, _CT_PAD, 128)
    ef_t = edge_feats.T
    b_col = b.reshape(_D_OUT, 1)
    out_t = pl.pallas_call(
        _edge_body,
        grid=(_GRID_B,),
        in_specs=[
            pl.BlockSpec((_D_OUT, _D_EDGE), lambda i: (0, 0)),
            pl.BlockSpec((_D_OUT, 128 * _CTB), lambda i: (0, i)),
            pl.BlockSpec((_D_OUT, _CTB, 128), lambda i: (0, i, 0)),
            pl.BlockSpec((_D_OUT, 1), lambda i: (0, 0)),
        ],
        out_specs=pl.BlockSpec((_D_OUT, 128 * _CTB), lambda i: (0, i)),
        out_shape=jax.ShapeDtypeStruct((_D_OUT, _N_EDGES), jnp.float32),
    )(wet, ef_t, g3, b_col)
    return out_t.T


# speedup vs baseline: 1.5024x; 1.0329x over previous
import jax
import jax.numpy as jnp
from jax import lax
from jax.experimental import pallas as pl
from jax.experimental.pallas import tpu as pltpu
from jax.experimental.pallas import tpu_sc as plsc

_N_NODES = 10000
_N_EDGES = 320000
_D_FEAT = 128
_D_EDGE = 16
_D_OUT = 16

_NC, _NS = 2, 16
_NW = _NC * _NS
_EPW = _N_EDGES // _NW
_CHUNK = 1000
_NCHUNK = _EPW // _CHUNK

_CT = _N_EDGES // 128
_CT_PAD = 2560
_N_PAD = _CT_PAD * 128
_CTB = 256
_GRID_B = 10


def _nodeproj_body(nf_ref, ws_ref, wd_ref, ps_ref, pd_ref):
    nf = nf_ref[...]
    ps_ref[...] = jnp.dot(nf, ws_ref[...], preferred_element_type=jnp.float32)
    pd_ref[...] = jnp.dot(nf, wd_ref[...], preferred_element_type=jnp.float32)


def _sc_gather_body(ps_hbm, pd_hbm, si_hbm, di_hbm, gt_hbm,
                    idx_s0, idx_s1, idx_d0, idx_d1,
                    buf_a0, buf_a1, buf_b0, buf_b1, buf_t,
                    sem_i0, sem_i1, sem_a0, sem_a1, sem_b0, sem_b1):
    wid = lax.axis_index("s") * _NC + lax.axis_index("c")
    base = wid * _EPW
    lanes = jnp.arange(16, dtype=jnp.int32)
    idx_s, idx_d = [idx_s0, idx_s1], [idx_d0, idx_d1]
    buf_a, buf_b = [buf_a0, buf_a1], [buf_b0, buf_b1]
    sem_i, sem_a, sem_b = [sem_i0, sem_i1], [sem_a0, sem_a1], [sem_b0, sem_b1]

    def start_idx(c, p):
        off = base + c * _CHUNK
        ci = pltpu.async_copy(si_hbm.at[pl.ds(off, _CHUNK)], idx_s[p], sem_i[p])
        cd = pltpu.async_copy(di_hbm.at[pl.ds(off, _CHUNK)], idx_d[p], sem_i[p])
        return ci, cd

    def start_gathers(p):
        ga = pltpu.async_copy(ps_hbm.at[idx_s[p]], buf_a[p], sem_a[p])
        gb = pltpu.async_copy(pd_hbm.at[idx_d[p]], buf_b[p], sem_b[p])
        return ga, gb

    i_cur = start_idx(0, 0)
    i_cur[0].wait()
    i_cur[1].wait()
    g = [None, None]
    g[0] = start_gathers(0)
    for c in range(_NCHUNK):
        p = c & 1
        if c + 1 < _NCHUNK:
            i_nxt = start_idx(c + 1, p ^ 1)
        g[p][0].wait()
        g[p][1].wait()
        if c + 1 < _NCHUNK:
            i_nxt[0].wait()
            i_nxt[1].wait()
            g[p ^ 1] = start_gathers(p ^ 1)
        ba, bb = buf_a[p], buf_b[p]

        def row_body(r, carry2, ba=ba, bb=bb):
            r4 = r * 4
            for k in range(4):
                v = ba[r4 + k, :] + bb[r4 + k, :]
                plsc.store_scatter(
                    buf_t, [lanes, jnp.full((16,), 0, jnp.int32) + (r4 + k)], v)
            return carry2

        lax.fori_loop(0, _CHUNK // 4, row_body, 0)
        off = base + c * _CHUNK
        pltpu.sync_copy(buf_t, gt_hbm.at[:, pl.ds(off, _CHUNK)])


def _edge_body(wet_ref, ef_ref, g_ref, b_ref, o_ref):
    wet = wet_ref[...]
    bb = b_ref[...]
    mm = jnp.dot(wet, ef_ref[...], preferred_element_type=jnp.float32) + bb
    for t in range(_CTB):
        o_ref[:, pl.ds(t * 128, 128)] = (
            mm[:, t * 128:(t + 1) * 128] + g_ref[:, t, :])


def kernel(node_feats, edge_feats, edge_indices, W, b):
    Wt = W.T
    wet = Wt[:, :_D_EDGE]
    Ws = W[_D_EDGE:_D_EDGE + _D_FEAT]
    Wd = W[_D_EDGE + _D_FEAT:]

    ps, pd = pl.pallas_call(
        _nodeproj_body,
        out_shape=[jax.ShapeDtypeStruct((_N_NODES, _D_OUT), jnp.float32)] * 2,
    )(node_feats, Ws, Wd)

    sc_gather = pl.kernel(
        _sc_gather_body,
        out_type=jax.ShapeDtypeStruct((_D_OUT, _N_PAD), jnp.float32),
        mesh=plsc.VectorSubcoreMesh(core_axis_name="c", subcore_axis_name="s"),
        scratch_types=(
            [pltpu.VMEM((_CHUNK,), jnp.int32)] * 4
            + [pltpu.VMEM((_CHUNK, _D_OUT), jnp.float32)] * 4
            + [pltpu.VMEM((_D_OUT, _CHUNK), jnp.float32)]
            + [pltpu.SemaphoreType.DMA] * 6
        ),
        compiler_params=pltpu.CompilerParams(use_tc_tiling_on_sc=False,
                                             needs_layout_passes=False),
    )
    gt = sc_gather(ps, pd, edge_indices[0], edge_indices[1])

    g3 = gt.reshape(_D_OUT, _CT_PAD, 128)
    ef_t = edge_feats.T
    b_col = b.reshape(_D_OUT, 1)
    out_t = pl.pallas_call(
        _edge_body,
        grid=(_GRID_B,),
        in_specs=[
            pl.BlockSpec((_D_OUT, _D_EDGE), lambda i: (0, 0)),
            pl.BlockSpec((_D_OUT, 128 * _CTB), lambda i: (0, i)),
            pl.BlockSpec((_D_OUT, _CTB, 128), lambda i: (0, i, 0)),
            pl.BlockSpec((_D_OUT, 1), lambda i: (0, 0)),
        ],
        out_specs=pl.BlockSpec((_D_OUT, 128 * _CTB), lambda i: (0, i)),
        out_shape=jax.ShapeDtypeStruct((_D_OUT, _N_EDGES), jnp.float32),
    )(wet, ef_t, g3, b_col)
    return out_t.T
